# Initial kernel scaffold; baseline (speedup 1.0000x reference)
import jax
import jax.numpy as jnp
from jax.experimental import pallas as pl


def kernel(text, context, mod_emb, rW1, rb1, rW2, rb2, rW3, rb3, in_proj_w, in_proj_b, out_w, out_b, gW1, gb1, gW2, gb2, eW1, eb1, eW2, eb2, eG, eB):
    raise NotImplementedError("write your pallas kernel here")


# speedup vs baseline: 6.1017x; 6.1017x over previous
import jax
import jax.numpy as jnp
from jax.experimental import pallas as pl
from jax.experimental.pallas import tpu as pltpu

D = 768
H = 12
DH = D // H
E = 8
K = 3
S = 2048
FCH = 4
FBLK = 2 * D // FCH


def _mmT(a, w):
    return jax.lax.dot_general(a, w, (((1,), (1,)), ((), ())),
                               preferred_element_type=jnp.float32)


def _stage1(text_ref, ctx_ref, mod0_ref,
            rW1_ref, rb1_ref, rW2_ref, rb2_ref, rW3_ref, rb3_ref,
            in_proj_w_ref, in_proj_b_ref,
            out_w_ref, out_b_ref,
            gW1_ref, gb1_ref, gW2_ref, gb2_ref,
            f_ref, routing_ref, wsel_ref, isel_ref):
    text = text_ref[...]
    mod0 = mod0_ref[...]
    enh = text + mod0

    summary = jnp.mean(enh, axis=0, keepdims=True)
    h1 = (_mmT(summary, rW1_ref[:, 0:D])
          + _mmT(ctx_ref[...], rW1_ref[:, D:2 * D]))
    h1 = jnp.maximum(h1 + rb1_ref[...], 0.0)
    h2 = jnp.maximum(_mmT(h1, rW2_ref[...]) + rb2_ref[...], 0.0)
    prod = h2 * rW3_ref[...]
    ones_mat = jnp.ones((D // 2, 128), jnp.float32)
    lg = jax.lax.dot_general(prod, ones_mat, (((1,), (0,)), ((), ())),
                             preferred_element_type=jnp.float32)
    rw = jax.nn.sigmoid(lg + rb3_ref[...])
    routing_ref[...] = rw / rw

    q0 = _mmT(enh[0:1], in_proj_w_ref[0:D, :]) + in_proj_b_ref[:, 0:D]
    h_iota = jax.lax.broadcasted_iota(jnp.int32, (H, D), 0)
    j_iota = jax.lax.broadcasted_iota(jnp.int32, (H, D), 1)
    head_mask = (j_iota // DH) == h_iota
    q_rows = jnp.where(head_mask, q0, 0.0)
    U = jax.lax.dot_general(q_rows, in_proj_w_ref[D:2 * D, :],
                            (((1,), (0,)), ((), ())),
                            preferred_element_type=jnp.float32)
    scores = _mmT(enh, U) * (1.0 / jnp.sqrt(jnp.float32(DH)))
    probs = jax.nn.softmax(scores, axis=0)
    P = jax.lax.dot_general(probs, enh, (((0,), (0,)), ((), ())),
                            preferred_element_type=jnp.float32)
    O = _mmT(P, in_proj_w_ref[2 * D:3 * D, :])
    o0 = jnp.sum(jnp.where(head_mask, O, 0.0), axis=0, keepdims=True)
    o0 = o0 + in_proj_b_ref[:, 2 * D:3 * D]
    f = _mmT(o0, out_w_ref[...]) + out_b_ref[...]
    f_ref[...] = f

    g1 = jnp.maximum(_mmT(f, gW1_ref[...]) + gb1_ref[...], 0.0)
    g = _mmT(g1, gW2_ref[...]) + gb2_ref[...]
    gp = jax.nn.softmax(g, axis=1)

    iota_e = jax.lax.broadcasted_iota(jnp.int32, (1, E), 1)

    def pick(vals):
        m = jnp.max(vals, axis=1, keepdims=True)
        idx = jnp.min(jnp.where(vals == m, iota_e, E), axis=1, keepdims=True)
        return m, idx

    m1, i1 = pick(gp)
    v1 = jnp.where(iota_e == i1, -jnp.inf, gp)
    m2, i2 = pick(v1)
    v2 = jnp.where(iota_e == i2, -jnp.inf, v1)
    m3, i3 = pick(v2)
    e1 = jnp.exp(m1 - m1)
    e2 = jnp.exp(m2 - m1)
    e3 = jnp.exp(m3 - m1)
    tot = e1 + e2 + e3
    wsel_ref[...] = (jnp.where(iota_e == 0, e1 / tot, 0.0)
                     + jnp.where(iota_e == 1, e2 / tot, 0.0)
                     + jnp.where(iota_e == 2, e3 / tot, 0.0))
    isel_ref[...] = (jnp.where(iota_e == 0, i1, 0)
                     + jnp.where(iota_e == 1, i2, 0)
                     + jnp.where(iota_e == 2, i3, 0))


def _stage2(idx_ref, f_ref, wsel_ref,
            eW1_ref, eb1_ref, eW2_ref, eb2_ref, eG_ref, eB_ref,
            out_ref, acc_ref):
    e = pl.program_id(0)
    c = pl.program_id(1)
    f = f_ref[...]
    eh = _mmT(f, eW1_ref[0]) + eb1_ref[0]
    eh = 0.5 * eh * (1.0 + jax.lax.erf(eh * (1.0 / jnp.sqrt(jnp.float32(2.0)))))
    part = _mmT(eh, eW2_ref[0])

    @pl.when(c == 0)
    def _():
        acc_ref[...] = jnp.zeros_like(acc_ref)

    acc_ref[...] += part

    @pl.when(c == FCH - 1)
    def _():
        eo = acc_ref[...] + eb2_ref[0]
        mu = jnp.mean(eo, axis=1, keepdims=True)
        cc = eo - mu
        var = jnp.mean(cc * cc, axis=1, keepdims=True)
        ln = cc / jnp.sqrt(var + 1e-5) * eG_ref[0] + eB_ref[0]
        iota_e = jax.lax.broadcasted_iota(jnp.int32, (1, E), 1)
        w = jnp.sum(jnp.where(iota_e == e, wsel_ref[...], 0.0))

        @pl.when(e == 0)
        def _():
            out_ref[...] = jnp.zeros_like(out_ref)

        out_ref[...] += w * ln


def kernel(text, context, mod_emb, rW1, rb1, rW2, rb2, rW3, rb3,
           in_proj_w, in_proj_b, out_w, out_b,
           gW1, gb1, gW2, gb2, eW1, eb1, eW2, eb2, eG, eB):
    ctx = context.reshape(1, D)
    mod0 = mod_emb[0:1, :]

    f, routing, wsel, isel = pl.pallas_call(
        _stage1,
        out_shape=(
            jax.ShapeDtypeStruct((1, D), jnp.float32),
            jax.ShapeDtypeStruct((1, 128), jnp.float32),
            jax.ShapeDtypeStruct((1, E), jnp.float32),
            jax.ShapeDtypeStruct((1, E), jnp.int32),
        ),
    )(text, ctx, mod0,
      rW1, rb1.reshape(1, D),
      rW2, rb2.reshape(1, D // 2), rW3,
      jnp.broadcast_to(rb3.reshape(1, 1), (1, 128)),
      in_proj_w, in_proj_b.reshape(1, 3 * D),
      out_w, out_b.reshape(1, D),
      gW1, gb1.reshape(1, D // 2), gW2, gb2.reshape(1, E))

    top_i = isel[0, :K]

    fused = pl.pallas_call(
        _stage2,
        grid_spec=pltpu.PrefetchScalarGridSpec(
            num_scalar_prefetch=1,
            grid=(K, FCH),
            in_specs=[
                pl.BlockSpec((1, D), lambda e, c, idx: (0, 0)),
                pl.BlockSpec((1, E), lambda e, c, idx: (0, 0)),
                pl.BlockSpec((1, FBLK, D), lambda e, c, idx: (idx[e], c, 0)),
                pl.BlockSpec((1, 1, FBLK), lambda e, c, idx: (idx[e], 0, c)),
                pl.BlockSpec((1, D, FBLK), lambda e, c, idx: (idx[e], 0, c)),
                pl.BlockSpec((1, 1, D), lambda e, c, idx: (idx[e], 0, 0)),
                pl.BlockSpec((1, 1, D), lambda e, c, idx: (idx[e], 0, 0)),
                pl.BlockSpec((1, 1, D), lambda e, c, idx: (idx[e], 0, 0)),
            ],
            out_specs=pl.BlockSpec((1, D), lambda e, c, idx: (0, 0)),
            scratch_shapes=[pltpu.VMEM((1, D), jnp.float32)],
        ),
        out_shape=jax.ShapeDtypeStruct((1, D), jnp.float32),
        compiler_params=pltpu.CompilerParams(
            dimension_semantics=("arbitrary", "arbitrary")),
    )(top_i, f, wsel,
      eW1, eb1.reshape(E, 1, 2 * D), eW2,
      eb2.reshape(E, 1, D), eG.reshape(E, 1, D), eB.reshape(E, 1, D))

    return fused.reshape(D), routing[0, 0]
